# baseline (speedup 1.0000x reference)
import functools

import jax
import jax.numpy as jnp
from jax import lax
from jax.experimental import pallas as pl
from jax.experimental.pallas import tpu as pltpu, tpu_sc as plsc

N, B, L, D, H = 8192, 8, 2048, 512, 1024
R = 1024
NB = N // R
NC, NS = 2, 16
NW = NC * NS
CHUNK = N // NW
GW = 128


NPAD = 2048
_C1 = 0.7978845608028654
_C2 = _C1 * 0.044715


def _gelu2(x):
    u = x * (_C1 + _C2 * (x * x))
    return x * (1.0 + jnp.tanh(u))


R2 = 2 * R
NBP = N // R2


def _table_body(x_ref, wff_ref, bff_ref, ws1_ref, bs1_ref, ws2_ref, bs2_ref,
                wff2_ref, bff2_ref, out_ref, wc1_s, bc1_s, wc2_s, bc2_s, pad_s):
    t = pl.program_id(0)
    f32 = jnp.float32
    bf16 = jnp.bfloat16

    @pl.when(t == 0)
    def _():
        wc1 = jnp.dot(wff_ref[...], ws1_ref[...], preferred_element_type=f32)
        wc1_s[...] = wc1.astype(bf16)
        bc1_s[...] = jnp.dot(bff_ref[...], ws1_ref[...],
                             preferred_element_type=f32) + bs1_ref[...]
        wc2 = jnp.dot(ws2_ref[...], wff2_ref[...], preferred_element_type=f32)
        wc2_s[...] = (0.5 * wc2).astype(bf16)
        bc2_s[...] = jnp.dot(bs2_ref[...], wff2_ref[...],
                             preferred_element_type=f32) + bff2_ref[...]
        pad_s[...] = (jnp.dot(_gelu2(bs1_ref[...]).astype(bf16), wc2_s[...],
                              preferred_element_type=f32) + bc2_s[...])

    @pl.when(t < NBP)
    def _():
        x0 = x_ref[0:R, :].astype(bf16)
        x1 = x_ref[R:R2, :].astype(bf16)
        pre0 = jnp.dot(x0, wc1_s[...], preferred_element_type=f32) + bc1_s[...]
        ga0 = _gelu2(pre0).astype(bf16)
        pre1 = jnp.dot(x1, wc1_s[...], preferred_element_type=f32) + bc1_s[...]
        ga1 = _gelu2(pre1).astype(bf16)
        out_ref[0:R, :] = (jnp.dot(ga0, wc2_s[...],
                                   preferred_element_type=f32) + bc2_s[...])
        out_ref[R:R2, :] = (jnp.dot(ga1, wc2_s[...],
                                    preferred_element_type=f32) + bc2_s[...])

    @pl.when(t >= NBP)
    def _():
        out_ref[...] = jnp.broadcast_to(pad_s[...], (R2, D))


def _table(in_feats, W_ff, b_ff, W_s1, b_s1, W_s2, b_s2, W_ff2, b_ff2):
    full = lambda r, c: pl.BlockSpec((r, c), lambda i: (0, 0))
    f32 = jnp.float32
    return pl.pallas_call(
        _table_body,
        grid=(NBP + NPAD // R2,),
        in_specs=[
            pl.BlockSpec((R2, D), lambda i: (jnp.minimum(i, NBP - 1), 0)),
            full(D, D), full(1, D), full(D, H), full(1, H),
            full(H, D), full(1, D), full(D, D), full(1, D),
        ],
        out_specs=pl.BlockSpec((R2, D), lambda i: (i, 0)),
        out_shape=jax.ShapeDtypeStruct((N + NPAD, D), f32),
        scratch_shapes=[
            pltpu.VMEM((D, H), jnp.bfloat16),
            pltpu.VMEM((1, H), f32),
            pltpu.VMEM((H, D), jnp.bfloat16),
            pltpu.VMEM((1, D), f32),
            pltpu.VMEM((1, D), f32),
        ],
    )(in_feats, W_ff, b_ff.reshape(1, D), W_s1, b_s1.reshape(1, H),
      W_s2, b_s2.reshape(1, D), W_ff2, b_ff2.reshape(1, D))


GSUB = 32
NG = CHUNK // GSUB


NBUF = 6
DEPTH = 4


def _sc_gather_body(table_hbm, n2s_hbm, s2n_hbm, out_hbm,
                    sbv, spv, fidx, nidx, rows,
                    g0s, g1s, g2s, g3s, g4s, g5s, w0s, w1s, w2s, w3s, w4s, w5s):
    gsems = (g0s, g1s, g2s, g3s, g4s, g5s)
    wsems = (w0s, w1s, w2s, w3s, w4s, w5s)
    wid = lax.axis_index("s") * NC + lax.axis_index("c")
    base = wid * CHUNK
    cb = pltpu.async_copy(s2n_hbm.at[0, pl.ds(base, CHUNK)], sbv, gsems[0])
    cp = pltpu.async_copy(s2n_hbm.at[1, pl.ds(base, CHUNK)], spv, gsems[1])
    cb.wait()
    cp.wait()

    def fidx_half(j):
        for o in range(GW // 16):
            k = j * (GW // 16) + o
            sl = pl.ds(k * 16, 16)
            fidx[j, pl.ds(o * 16, 16)] = sbv[sl] * L + spv[sl]

    fidx_half(0)
    g0 = pltpu.async_copy(n2s_hbm.at[fidx.at[0]], nidx.at[0], gsems[0])
    fidx_half(1)
    g1 = pltpu.async_copy(n2s_hbm.at[fidx.at[1]], nidx.at[1], gsems[1])

    lane = lax.iota(jnp.int32, 16)

    def remap_half(j):
        for o in range(GW // 16):
            sl = pl.ds(o * 16, 16)
            v = nidx[j, sl]
            padrow = N + ((base + (j * (GW // 16) + o) * 16 + lane) & (NPAD - 1))
            nidx[j, sl] = jnp.where(v < 0, padrow, v)

    gcopy, wcopy = [None] * NBUF, [None] * NBUF

    def start_gather(g):
        b = g % NBUF
        j, o = divmod(g, GW // GSUB)
        idx = nidx.at[j, pl.ds(o * GSUB, GSUB)]
        gcopy[b] = pltpu.async_copy(table_hbm.at[idx], rows.at[b], gsems[b])

    g0.wait()
    remap_half(0)
    for g in range(DEPTH):
        start_gather(g)
    g1.wait()
    remap_half(1)
    for g in range(NG):
        if g + DEPTH < NG:
            b2 = (g + DEPTH) % NBUF
            if wcopy[b2] is not None:
                wcopy[b2].wait()
            start_gather(g + DEPTH)
        b = g % NBUF
        gcopy[b].wait()
        wcopy[b] = pltpu.async_copy(
            rows.at[b], out_hbm.at[pl.ds(base + g * GSUB, GSUB)], wsems[b])
    for b in range(NBUF):
        if wcopy[b] is not None:
            wcopy[b].wait()


def _sc_gather(table_ext, n2s_flat, seq2node):
    mesh = plsc.VectorSubcoreMesh(core_axis_name="c", subcore_axis_name="s")
    f = pl.kernel(
        _sc_gather_body,
        out_type=jax.ShapeDtypeStruct((N, D), jnp.float32),
        mesh=mesh,
        scratch_types=[
            pltpu.VMEM((CHUNK,), jnp.int32),
            pltpu.VMEM((CHUNK,), jnp.int32),
            pltpu.VMEM((CHUNK // GW, GW), jnp.int32),
            pltpu.VMEM((CHUNK // GW, GW), jnp.int32),
            pltpu.VMEM((NBUF, GSUB, D), jnp.float32),
            pltpu.SemaphoreType.DMA,
            pltpu.SemaphoreType.DMA,
            pltpu.SemaphoreType.DMA,
            pltpu.SemaphoreType.DMA,
            pltpu.SemaphoreType.DMA,
            pltpu.SemaphoreType.DMA,
            pltpu.SemaphoreType.DMA,
            pltpu.SemaphoreType.DMA,
            pltpu.SemaphoreType.DMA,
            pltpu.SemaphoreType.DMA,
            pltpu.SemaphoreType.DMA,
            pltpu.SemaphoreType.DMA,
        ],
    )
    return f(table_ext, n2s_flat, seq2node)


def kernel(graph, in_feats, node2seq, seq2node, W_ff, b_ff, W_s1, b_s1,
           W_s2, b_s2, W_ff2, b_ff2):
    table_ext = _table(in_feats, W_ff, b_ff, W_s1, b_s1, W_s2, b_s2,
                       W_ff2, b_ff2)
    n2s_flat = node2seq.reshape(-1).astype(jnp.int32)
    return _sc_gather(table_ext, n2s_flat, seq2node.astype(jnp.int32))


# speedup vs baseline: 1.0039x; 1.0039x over previous
import jax
import jax.numpy as jnp
from jax import lax
from jax.experimental import pallas as pl
from jax.experimental.pallas import tpu as pltpu, tpu_sc as plsc

N, B, L, D, H = 8192, 8, 2048, 512, 1024
R = 1024
NB = N // R
NC, NS = 2, 16
NW = NC * NS
CHUNK = N // NW
GW = 128


NPAD = 1024
_C1 = 0.7978845608028654
_C2 = _C1 * 0.044715


def _gelu2(x):
    u = x * (_C1 + _C2 * (x * x))
    return x * (1.0 + jnp.tanh(u))


def _table_body(x_ref, wff_ref, bff_ref, ws1_ref, bs1_ref, ws2_ref, bs2_ref,
                wff2_ref, bff2_ref, out_ref, wc1_s, bc1_s, wc2_s, bc2_s, pad_s):
    i = pl.program_id(0)
    f32 = jnp.float32
    bf16 = jnp.bfloat16

    @pl.when(i == 0)
    def _():
        wc1 = jnp.dot(wff_ref[...], ws1_ref[...], preferred_element_type=f32)
        wc1_s[...] = wc1.astype(bf16)
        bc1_s[...] = jnp.dot(bff_ref[...], ws1_ref[...],
                             preferred_element_type=f32) + bs1_ref[...]
        wc2 = jnp.dot(ws2_ref[...], wff2_ref[...], preferred_element_type=f32)
        wc2_s[...] = (0.5 * wc2).astype(bf16)
        bc2_s[...] = jnp.dot(bs2_ref[...], wff2_ref[...],
                             preferred_element_type=f32) + bff2_ref[...]
        pad_s[...] = (jnp.dot(_gelu2(bs1_ref[...]).astype(bf16), wc2_s[...],
                              preferred_element_type=f32) + bc2_s[...])

    @pl.when(i < NB)
    def _():
        pre = jnp.dot(x_ref[...].astype(bf16), wc1_s[...],
                      preferred_element_type=f32) + bc1_s[...]
        out_ref[...] = (jnp.dot(_gelu2(pre).astype(bf16), wc2_s[...],
                                preferred_element_type=f32) + bc2_s[...])

    @pl.when(i >= NB)
    def _():
        out_ref[...] = jnp.broadcast_to(pad_s[...], (R, D))


def _table(in_feats, W_ff, b_ff, W_s1, b_s1, W_s2, b_s2, W_ff2, b_ff2):
    full = lambda r, c: pl.BlockSpec((r, c), lambda i: (0, 0))
    f32 = jnp.float32
    return pl.pallas_call(
        _table_body,
        grid=(NB + NPAD // R,),
        in_specs=[
            pl.BlockSpec((R, D), lambda i: (jnp.minimum(i, NB - 1), 0)),
            full(D, D), full(1, D), full(D, H), full(1, H),
            full(H, D), full(1, D), full(D, D), full(1, D),
        ],
        out_specs=pl.BlockSpec((R, D), lambda i: (i, 0)),
        out_shape=jax.ShapeDtypeStruct((N + NPAD, D), f32),
        scratch_shapes=[
            pltpu.VMEM((D, H), jnp.bfloat16),
            pltpu.VMEM((1, H), f32),
            pltpu.VMEM((H, D), jnp.bfloat16),
            pltpu.VMEM((1, D), f32),
            pltpu.VMEM((1, D), f32),
        ],
    )(in_feats, W_ff, b_ff.reshape(1, D), W_s1, b_s1.reshape(1, H),
      W_s2, b_s2.reshape(1, D), W_ff2, b_ff2.reshape(1, D))


GSUB = 32
NG = CHUNK // GSUB


NBUF = 6
DEPTH = 4


def _sc_gather_body(table_hbm, n2s_hbm, s2n_hbm, out_hbm,
                    sbv, spv, fidx, nidx, rows,
                    g0s, g1s, g2s, g3s, g4s, g5s, w0s, w1s, w2s, w3s, w4s, w5s):
    gsems = (g0s, g1s, g2s, g3s, g4s, g5s)
    wsems = (w0s, w1s, w2s, w3s, w4s, w5s)
    wid = lax.axis_index("s") * NC + lax.axis_index("c")
    base = wid * CHUNK
    cb = pltpu.async_copy(s2n_hbm.at[0, pl.ds(base, CHUNK)], sbv, gsems[0])
    cp = pltpu.async_copy(s2n_hbm.at[1, pl.ds(base, CHUNK)], spv, gsems[1])
    cb.wait()
    cp.wait()

    def fidx_half(j):
        for o in range(GW // 16):
            k = j * (GW // 16) + o
            sl = pl.ds(k * 16, 16)
            fidx[j, pl.ds(o * 16, 16)] = sbv[sl] * L + spv[sl]

    fidx_half(0)
    g0 = pltpu.async_copy(n2s_hbm.at[fidx.at[0]], nidx.at[0], gsems[0])
    fidx_half(1)
    g1 = pltpu.async_copy(n2s_hbm.at[fidx.at[1]], nidx.at[1], gsems[1])

    lane = lax.iota(jnp.int32, 16)

    def remap_half(j):
        for o in range(GW // 16):
            sl = pl.ds(o * 16, 16)
            v = nidx[j, sl]
            padrow = N + ((base + (j * (GW // 16) + o) * 16 + lane) & (NPAD - 1))
            nidx[j, sl] = jnp.where(v < 0, padrow, v)

    gcopy, wcopy = [None] * NBUF, [None] * NBUF

    def start_gather(g):
        b = g % NBUF
        j, o = divmod(g, GW // GSUB)
        idx = nidx.at[j, pl.ds(o * GSUB, GSUB)]
        gcopy[b] = pltpu.async_copy(table_hbm.at[idx], rows.at[b], gsems[b])

    g0.wait()
    remap_half(0)
    for g in range(DEPTH):
        start_gather(g)
    g1.wait()
    remap_half(1)
    for g in range(NG):
        if g + DEPTH < NG:
            b2 = (g + DEPTH) % NBUF
            if wcopy[b2] is not None:
                wcopy[b2].wait()
            start_gather(g + DEPTH)
        b = g % NBUF
        gcopy[b].wait()
        wcopy[b] = pltpu.async_copy(
            rows.at[b], out_hbm.at[pl.ds(base + g * GSUB, GSUB)], wsems[b])
    for b in range(NBUF):
        if wcopy[b] is not None:
            wcopy[b].wait()


def _sc_gather(table_ext, n2s_flat, seq2node):
    mesh = plsc.VectorSubcoreMesh(core_axis_name="c", subcore_axis_name="s")
    f = pl.kernel(
        _sc_gather_body,
        out_type=jax.ShapeDtypeStruct((N, D), jnp.float32),
        mesh=mesh,
        scratch_types=[
            pltpu.VMEM((CHUNK,), jnp.int32),
            pltpu.VMEM((CHUNK,), jnp.int32),
            pltpu.VMEM((CHUNK // GW, GW), jnp.int32),
            pltpu.VMEM((CHUNK // GW, GW), jnp.int32),
            pltpu.VMEM((NBUF, GSUB, D), jnp.float32),
            pltpu.SemaphoreType.DMA,
            pltpu.SemaphoreType.DMA,
            pltpu.SemaphoreType.DMA,
            pltpu.SemaphoreType.DMA,
            pltpu.SemaphoreType.DMA,
            pltpu.SemaphoreType.DMA,
            pltpu.SemaphoreType.DMA,
            pltpu.SemaphoreType.DMA,
            pltpu.SemaphoreType.DMA,
            pltpu.SemaphoreType.DMA,
            pltpu.SemaphoreType.DMA,
            pltpu.SemaphoreType.DMA,
        ],
    )
    return f(table_ext, n2s_flat, seq2node)


def kernel(graph, in_feats, node2seq, seq2node, W_ff, b_ff, W_s1, b_s1,
           W_s2, b_s2, W_ff2, b_ff2):
    table_ext = _table(in_feats, W_ff, b_ff, W_s1, b_s1, W_s2, b_s2,
                       W_ff2, b_ff2)
    n2s_flat = node2seq.reshape(-1).astype(jnp.int32)
    return _sc_gather(table_ext, n2s_flat, seq2node.astype(jnp.int32))
